# baseline (speedup 1.0000x reference)
import functools

import jax
import jax.numpy as jnp
from jax.experimental import pallas as pl
from jax.experimental.pallas import tpu as pltpu

_BN_EPS = 1e-5
_N1, _N2, _N3 = 1024, 512, 256
_CA = 40
_VMEM = 64 * 1024 * 1024


def _moments_kernel(x_ref, s_ref, xa_ref):
    t = pl.program_id(1)

    @pl.when(t == 0)
    def _():
        s_ref[...] = jnp.zeros_like(s_ref)

    xb = x_ref[...].astype(jnp.bfloat16)
    ones = jnp.ones((1, xb.shape[1]), jnp.bfloat16)
    zeros = jnp.zeros((7, xb.shape[1]), jnp.bfloat16)
    xa = jnp.concatenate([xb, ones, zeros], axis=0)
    xa_ref[...] = xa
    s_ref[0] += jax.lax.dot_general(
        xa, xa, (((1,), (1,)), ((), ())),
        preferred_element_type=jnp.float32)


def _mid_kernel(xa_ref, s1_ref, w1fp_ref, g1_ref, bt1_ref, w2_ref,
                y2_ref, ss_ref, sq_ref, w1s_ref, accs_ref, accq_ref,
                *, m_total, eps):
    t = pl.program_id(1)
    n_t = pl.num_programs(1)

    @pl.when(t == 0)
    def _():
        parts = s1_ref[0] + s1_ref[1]
        w1fp = w1fp_ref[...]
        inv_m = 1.0 / m_total
        msum = parts[32:33, :]
        es = jnp.sum(w1fp * (msum * inv_m), axis=1, keepdims=True)
        u = jax.lax.dot(w1fp, parts,
                        precision=jax.lax.Precision.HIGHEST,
                        preferred_element_type=jnp.float32)
        q = jnp.sum(u * w1fp, axis=1, keepdims=True)
        var = jnp.maximum(q * inv_m - es * es, 0.0)
        a1 = g1_ref[...] * jax.lax.rsqrt(var + eps)
        c1 = bt1_ref[...] - a1 * es
        lane = jax.lax.broadcasted_iota(jnp.int32, (_N1, _CA), 1)
        w1s_ref[...] = jnp.where(lane == 32, c1, w1fp * a1).astype(jnp.bfloat16)

    y1 = jnp.dot(w1s_ref[...], xa_ref[...],
                 preferred_element_type=jnp.float32)
    h1 = jnp.maximum(y1, 0.0).astype(jnp.bfloat16)
    y2 = jnp.dot(w2_ref[...], h1, preferred_element_type=jnp.float32)
    y2_ref[...] = y2.astype(jnp.bfloat16)

    @pl.when(t == 0)
    def _():
        accs_ref[...] = y2
        accq_ref[...] = y2 * y2

    @pl.when(t > 0)
    def _():
        accs_ref[...] += y2
        accq_ref[...] += y2 * y2

    @pl.when(t == n_t - 1)
    def _():
        ss_ref[0] = jnp.sum(accs_ref[...], axis=1, keepdims=True)
        sq_ref[0] = jnp.sum(accq_ref[...], axis=1, keepdims=True)


def _out_kernel(y2_ref, s2_ref, q2_ref, g2_ref, bt2_ref, w3_ref, b3_ref,
                o_ref, *, m_total, eps):
    inv_m = 1.0 / m_total
    ssum = s2_ref[0] + s2_ref[1]
    sqs = q2_ref[0] + q2_ref[1]
    mean = ssum * inv_m
    var = jnp.maximum(sqs * inv_m - mean * mean, 0.0)
    a2 = g2_ref[...] * jax.lax.rsqrt(var + eps)
    c2 = bt2_ref[...] - a2 * mean
    h2 = jnp.maximum(y2_ref[...].astype(jnp.float32) * a2 + c2,
                     0.0).astype(jnp.bfloat16)
    o_ref[...] = (jnp.dot(w3_ref[...], h2, preferred_element_type=jnp.float32)
                  + b3_ref[...])


def kernel(x, w1, b1, w2, b2, w3, b3, g1, beta1, g2, beta2):
    B, Cin, L = x.shape
    M = B * L
    TL = 1024 if L % 1024 == 0 else L
    n_tiles = M // TL
    nT = n_tiles // 2
    t_per_b = L // TL
    nb = B // 2

    x2 = x.reshape(B * Cin, L)

    w1fp = jnp.pad(w1.T, ((0, 0), (0, _CA - Cin)))
    w2t = w2.T.astype(jnp.bfloat16)
    w3t = w3.T.astype(jnp.bfloat16)
    g1c = g1.reshape(_N1, 1)
    bt1c = beta1.reshape(_N1, 1)
    g2c = g2.reshape(_N2, 1)
    bt2c = beta2.reshape(_N2, 1)
    b3c = b3.reshape(_N3, 1)

    s1, xpad = pl.pallas_call(
        _moments_kernel,
        out_shape=(jax.ShapeDtypeStruct((2, _CA, _CA), jnp.float32),
                   jax.ShapeDtypeStruct((B * _CA, L), jnp.bfloat16)),
        grid=(2, nb),
        in_specs=[pl.BlockSpec((Cin, L), lambda c, t: (c * nb + t, 0))],
        out_specs=(pl.BlockSpec((1, _CA, _CA), lambda c, t: (c, 0, 0)),
                   pl.BlockSpec((_CA, L), lambda c, t: (c * nb + t, 0))),
        compiler_params=pltpu.CompilerParams(
            dimension_semantics=("parallel", "arbitrary"),
            vmem_limit_bytes=_VMEM),
    )(x2)

    y2, ss2, sq2 = pl.pallas_call(
        functools.partial(_mid_kernel, m_total=float(M), eps=_BN_EPS),
        out_shape=(jax.ShapeDtypeStruct((_N2, M), jnp.bfloat16),
                   jax.ShapeDtypeStruct((2, _N2, 1), jnp.float32),
                   jax.ShapeDtypeStruct((2, _N2, 1), jnp.float32)),
        grid=(2, nT),
        in_specs=[
            pl.BlockSpec((_CA, TL),
                         lambda c, t: ((c * nT + t) // t_per_b,
                                       (c * nT + t) % t_per_b)),
            pl.BlockSpec((2, _CA, _CA), lambda c, t: (0, 0, 0)),
            pl.BlockSpec((_N1, _CA), lambda c, t: (0, 0)),
            pl.BlockSpec((_N1, 1), lambda c, t: (0, 0)),
            pl.BlockSpec((_N1, 1), lambda c, t: (0, 0)),
            pl.BlockSpec((_N2, _N1), lambda c, t: (0, 0)),
        ],
        out_specs=(pl.BlockSpec((_N2, TL), lambda c, t: (0, c * nT + t)),
                   pl.BlockSpec((1, _N2, 1), lambda c, t: (c, 0, 0)),
                   pl.BlockSpec((1, _N2, 1), lambda c, t: (c, 0, 0))),
        scratch_shapes=[pltpu.VMEM((_N1, _CA), jnp.bfloat16),
                        pltpu.VMEM((_N2, TL), jnp.float32),
                        pltpu.VMEM((_N2, TL), jnp.float32)],
        compiler_params=pltpu.CompilerParams(
            dimension_semantics=("parallel", "arbitrary"),
            vmem_limit_bytes=_VMEM),
    )(xpad, s1, w1fp, g1c, bt1c, w2t)

    o2 = pl.pallas_call(
        functools.partial(_out_kernel, m_total=float(M), eps=_BN_EPS),
        out_shape=jax.ShapeDtypeStruct((B * _N3, L), jnp.float32),
        grid=(2, nT),
        in_specs=[
            pl.BlockSpec((_N2, TL), lambda c, t: (0, c * nT + t)),
            pl.BlockSpec((2, _N2, 1), lambda c, t: (0, 0, 0)),
            pl.BlockSpec((2, _N2, 1), lambda c, t: (0, 0, 0)),
            pl.BlockSpec((_N2, 1), lambda c, t: (0, 0)),
            pl.BlockSpec((_N2, 1), lambda c, t: (0, 0)),
            pl.BlockSpec((_N3, _N2), lambda c, t: (0, 0)),
            pl.BlockSpec((_N3, 1), lambda c, t: (0, 0)),
        ],
        out_specs=pl.BlockSpec((_N3, TL),
                               lambda c, t: ((c * nT + t) // t_per_b,
                                             (c * nT + t) % t_per_b)),
        compiler_params=pltpu.CompilerParams(
            dimension_semantics=("parallel", "arbitrary"),
            vmem_limit_bytes=_VMEM),
    )(y2, ss2, sq2, g2c, bt2c, w3t, b3c)

    return o2.reshape(B, _N3, L)


# speedup vs baseline: 1.3398x; 1.3398x over previous
import functools

import jax
import jax.numpy as jnp
from jax.experimental import pallas as pl
from jax.experimental.pallas import tpu as pltpu

_BN_EPS = 1e-5
_N1, _N2, _N3 = 1024, 512, 256
_CA = 40
_BB = 4
_VMEM = 56 * 1024 * 1024


def _pre_kernel(x_ref, w1fp_ref, g1_ref, bt1_ref, xa_ref, w1s_ref, s_ref,
                *, m_total, eps):
    t = pl.program_id(0)
    n_t = pl.num_programs(0)
    ncols = x_ref.shape[1]

    acc = None
    for i in range(_BB):
        xb = x_ref[32 * i:32 * (i + 1), :].astype(jnp.bfloat16)
        xa = jnp.concatenate(
            [xb, jnp.ones((1, ncols), jnp.bfloat16),
             jnp.zeros((7, ncols), jnp.bfloat16)], axis=0)
        xa_ref[_CA * i:_CA * (i + 1), :] = xa
        p = jax.lax.dot_general(
            xa, xa, (((1,), (1,)), ((), ())),
            preferred_element_type=jnp.float32)
        acc = p if acc is None else acc + p

    @pl.when(t == 0)
    def _():
        s_ref[...] = acc

    @pl.when(t > 0)
    def _():
        s_ref[...] += acc

    @pl.when(t == n_t - 1)
    def _():
        parts = s_ref[...]
        w1fp = w1fp_ref[...]
        inv_m = 1.0 / m_total
        msum = parts[32:33, :]
        es = jnp.sum(w1fp * (msum * inv_m), axis=1, keepdims=True)
        u = jax.lax.dot(w1fp, parts,
                        precision=jax.lax.Precision.HIGHEST,
                        preferred_element_type=jnp.float32)
        q = jnp.sum(u * w1fp, axis=1, keepdims=True)
        var = jnp.maximum(q * inv_m - es * es, 0.0)
        a1 = g1_ref[...] * jax.lax.rsqrt(var + eps)
        c1 = bt1_ref[...] - a1 * es
        lane = jax.lax.broadcasted_iota(jnp.int32, (_N1, _CA), 1)
        w1s_ref[...] = jnp.where(lane == 32, c1,
                                 w1fp * a1).astype(jnp.bfloat16)


def _mid_kernel(xa_ref, w1s_ref, w2_ref, y2_ref, st_ref, accs_ref, accq_ref):
    t = pl.program_id(0)
    n_t = pl.num_programs(0)

    y1 = jnp.dot(w1s_ref[...], xa_ref[...],
                 preferred_element_type=jnp.float32)
    h1 = jnp.maximum(y1, 0.0).astype(jnp.bfloat16)
    y2 = jnp.dot(w2_ref[...], h1, preferred_element_type=jnp.float32)
    y2_ref[...] = y2.astype(jnp.bfloat16)

    q2 = y2 * y2
    tl = y2.shape[1]
    ps = y2[:, 0:128]
    pq = q2[:, 0:128]
    for off in range(128, tl, 128):
        ps = ps + y2[:, off:off + 128]
        pq = pq + q2[:, off:off + 128]

    @pl.when(t == 0)
    def _():
        accs_ref[...] = ps
        accq_ref[...] = pq

    @pl.when(t > 0)
    def _():
        accs_ref[...] += ps
        accq_ref[...] += pq

    @pl.when(t == n_t - 1)
    def _():
        st_ref[0:_N2] = jnp.sum(accs_ref[...], axis=1, keepdims=True)
        st_ref[_N2:2 * _N2] = jnp.sum(accq_ref[...], axis=1, keepdims=True)


def _out_kernel(y2_ref, st_ref, p_ref, w3_ref, o_ref, *, m_total, eps):
    inv_m = 1.0 / m_total
    mean = st_ref[0:_N2] * inv_m
    var = jnp.maximum(st_ref[_N2:2 * _N2] * inv_m - mean * mean, 0.0)
    a2 = p_ref[0:_N2] * jax.lax.rsqrt(var + eps)
    c2 = p_ref[_N2:2 * _N2] - a2 * mean
    b3 = p_ref[2 * _N2:2 * _N2 + _N3]
    h2 = jnp.maximum(y2_ref[...].astype(jnp.float32) * a2 + c2,
                     0.0).astype(jnp.bfloat16)
    o_ref[...] = (jnp.dot(w3_ref[...], h2, preferred_element_type=jnp.float32)
                  + b3)


def kernel(x, w1, b1, w2, b2, w3, b3, g1, beta1, g2, beta2):
    B, Cin, L = x.shape
    M = B * L
    TL = 2048 if L % 2048 == 0 else L
    n_tiles = M // TL
    t_per_b = L // TL

    x2 = x.reshape(B * Cin, L)

    w1fp = jnp.pad(w1.T, ((0, 0), (0, _CA - Cin)))
    w2t = w2.T.astype(jnp.bfloat16)
    w3t = w3.T.astype(jnp.bfloat16)
    g1c = g1.reshape(_N1, 1)
    bt1c = beta1.reshape(_N1, 1)
    pcol = jnp.concatenate(
        [g2, beta2, b3]).reshape(2 * _N2 + _N3, 1)

    xpad, w1s = pl.pallas_call(
        functools.partial(_pre_kernel, m_total=float(M), eps=_BN_EPS),
        out_shape=(jax.ShapeDtypeStruct((B * _CA, L), jnp.bfloat16),
                   jax.ShapeDtypeStruct((_N1, _CA), jnp.bfloat16)),
        grid=(B // _BB,),
        in_specs=[
            pl.BlockSpec((_BB * Cin, L), lambda t: (t, 0)),
            pl.BlockSpec((_N1, _CA), lambda t: (0, 0)),
            pl.BlockSpec((_N1, 1), lambda t: (0, 0)),
            pl.BlockSpec((_N1, 1), lambda t: (0, 0)),
        ],
        out_specs=(pl.BlockSpec((_BB * _CA, L), lambda t: (t, 0)),
                   pl.BlockSpec((_N1, _CA), lambda t: (0, 0))),
        scratch_shapes=[pltpu.VMEM((_CA, _CA), jnp.float32)],
        compiler_params=pltpu.CompilerParams(
            dimension_semantics=("arbitrary",),
            vmem_limit_bytes=_VMEM),
    )(x2, w1fp, g1c, bt1c)

    y2, st2 = pl.pallas_call(
        _mid_kernel,
        out_shape=(jax.ShapeDtypeStruct((_N2, M), jnp.bfloat16),
                   jax.ShapeDtypeStruct((2 * _N2, 1), jnp.float32)),
        grid=(n_tiles,),
        in_specs=[
            pl.BlockSpec((_CA, TL), lambda t: (t // t_per_b, t % t_per_b)),
            pl.BlockSpec((_N1, _CA), lambda t: (0, 0)),
            pl.BlockSpec((_N2, _N1), lambda t: (0, 0)),
        ],
        out_specs=(pl.BlockSpec((_N2, TL), lambda t: (0, t)),
                   pl.BlockSpec((2 * _N2, 1), lambda t: (0, 0))),
        scratch_shapes=[pltpu.VMEM((_N2, 128), jnp.float32),
                        pltpu.VMEM((_N2, 128), jnp.float32)],
        compiler_params=pltpu.CompilerParams(
            dimension_semantics=("arbitrary",),
            vmem_limit_bytes=_VMEM),
    )(xpad, w1s, w2t)

    o2 = pl.pallas_call(
        functools.partial(_out_kernel, m_total=float(M), eps=_BN_EPS),
        out_shape=jax.ShapeDtypeStruct((B * _N3, L), jnp.float32),
        grid=(n_tiles,),
        in_specs=[
            pl.BlockSpec((_N2, TL), lambda t: (0, t)),
            pl.BlockSpec((2 * _N2, 1), lambda t: (0, 0)),
            pl.BlockSpec((2 * _N2 + _N3, 1), lambda t: (0, 0)),
            pl.BlockSpec((_N3, _N2), lambda t: (0, 0)),
        ],
        out_specs=pl.BlockSpec((_N3, TL),
                               lambda t: (t // t_per_b, t % t_per_b)),
        compiler_params=pltpu.CompilerParams(
            dimension_semantics=("arbitrary",),
            vmem_limit_bytes=_VMEM),
    )(y2, st2, pcol, w3t)

    return o2.reshape(B, _N3, L)


# speedup vs baseline: 1.4649x; 1.0934x over previous
import functools

import jax
import jax.numpy as jnp
from jax.experimental import pallas as pl
from jax.experimental.pallas import tpu as pltpu

_BN_EPS = 1e-5
_N1, _N2, _N3 = 1024, 512, 256
_CA = 40
_BB = 4
_VMEM = 56 * 1024 * 1024


def _pre_kernel(x_ref, w1fp_ref, g1_ref, bt1_ref, xa_ref, w1s_ref, s_ref,
                *, m_total, eps):
    t = pl.program_id(0)
    n_t = pl.num_programs(0)
    ncols = x_ref.shape[1]

    acc = None
    for i in range(_BB):
        xb = x_ref[32 * i:32 * (i + 1), :].astype(jnp.bfloat16)
        xa = jnp.concatenate(
            [xb, jnp.ones((1, ncols), jnp.bfloat16),
             jnp.zeros((7, ncols), jnp.bfloat16)], axis=0)
        xa_ref[_CA * i:_CA * (i + 1), :] = xa
        p = jax.lax.dot_general(
            xa, xa, (((1,), (1,)), ((), ())),
            preferred_element_type=jnp.float32)
        acc = p if acc is None else acc + p

    @pl.when(t == 0)
    def _():
        s_ref[...] = acc

    @pl.when(t > 0)
    def _():
        s_ref[...] += acc

    @pl.when(t == n_t - 1)
    def _():
        parts = s_ref[...]
        w1fp = w1fp_ref[...]
        inv_m = 1.0 / m_total
        msum = parts[32:33, :]
        es = jnp.sum(w1fp * (msum * inv_m), axis=1, keepdims=True)
        u = jax.lax.dot(w1fp, parts,
                        precision=jax.lax.Precision.HIGHEST,
                        preferred_element_type=jnp.float32)
        q = jnp.sum(u * w1fp, axis=1, keepdims=True)
        var = jnp.maximum(q * inv_m - es * es, 0.0)
        a1 = g1_ref[...] * jax.lax.rsqrt(var + eps)
        c1 = bt1_ref[...] - a1 * es
        lane = jax.lax.broadcasted_iota(jnp.int32, (_N1, _CA), 1)
        w1s_ref[...] = jnp.where(lane == 32, c1,
                                 w1fp * a1).astype(jnp.bfloat16)


def _mid_kernel(xa_ref, w1s_ref, w2_ref, y2_ref, st_ref, accs_ref, accq_ref):
    t = pl.program_id(0)
    n_t = pl.num_programs(0)

    y1 = jnp.dot(w1s_ref[...], xa_ref[...],
                 preferred_element_type=jnp.float32)
    h1 = jnp.maximum(y1, 0.0).astype(jnp.bfloat16)
    y2 = jnp.dot(w2_ref[...], h1, preferred_element_type=jnp.float32)
    y2_ref[...] = y2.astype(jnp.bfloat16)

    q2 = y2 * y2
    tl = y2.shape[1]
    ps = y2[:, 0:128]
    pq = q2[:, 0:128]
    for off in range(128, tl, 128):
        ps = ps + y2[:, off:off + 128]
        pq = pq + q2[:, off:off + 128]

    @pl.when(t == 0)
    def _():
        accs_ref[...] = ps
        accq_ref[...] = pq

    @pl.when(t > 0)
    def _():
        accs_ref[...] += ps
        accq_ref[...] += pq

    @pl.when(t == n_t - 1)
    def _():
        st_ref[0:_N2] = jnp.sum(accs_ref[...], axis=1, keepdims=True)
        st_ref[_N2:2 * _N2] = jnp.sum(accq_ref[...], axis=1, keepdims=True)


def _out_kernel(y2_ref, st_ref, p_ref, w3_ref, o_ref, *, m_total, eps):
    inv_m = 1.0 / m_total
    mean = st_ref[0:_N2] * inv_m
    var = jnp.maximum(st_ref[_N2:2 * _N2] * inv_m - mean * mean, 0.0)
    a2 = p_ref[0:_N2] * jax.lax.rsqrt(var + eps)
    c2 = p_ref[_N2:2 * _N2] - a2 * mean
    b3 = p_ref[2 * _N2:2 * _N2 + _N3]
    h2 = jnp.maximum(y2_ref[...].astype(jnp.float32) * a2 + c2,
                     0.0).astype(jnp.bfloat16)
    o_ref[...] = (jnp.dot(w3_ref[...], h2, preferred_element_type=jnp.float32)
                  + b3)


def kernel(x, w1, b1, w2, b2, w3, b3, g1, beta1, g2, beta2):
    B, Cin, L = x.shape
    M = B * L
    TL = 4096 if L % 4096 == 0 else (2048 if L % 2048 == 0 else L)
    n_tiles = M // TL
    t_per_b = L // TL

    x2 = x.reshape(B * Cin, L)

    w1fp = jnp.pad(w1.T, ((0, 0), (0, _CA - Cin)))
    w2t = w2.T.astype(jnp.bfloat16)
    w3t = w3.T.astype(jnp.bfloat16)
    g1c = g1.reshape(_N1, 1)
    bt1c = beta1.reshape(_N1, 1)
    pcol = jnp.concatenate(
        [g2, beta2, b3]).reshape(2 * _N2 + _N3, 1)

    xpad, w1s = pl.pallas_call(
        functools.partial(_pre_kernel, m_total=float(M), eps=_BN_EPS),
        out_shape=(jax.ShapeDtypeStruct((B * _CA, L), jnp.bfloat16),
                   jax.ShapeDtypeStruct((_N1, _CA), jnp.bfloat16)),
        grid=(B // _BB,),
        in_specs=[
            pl.BlockSpec((_BB * Cin, L), lambda t: (t, 0)),
            pl.BlockSpec((_N1, _CA), lambda t: (0, 0)),
            pl.BlockSpec((_N1, 1), lambda t: (0, 0)),
            pl.BlockSpec((_N1, 1), lambda t: (0, 0)),
        ],
        out_specs=(pl.BlockSpec((_BB * _CA, L), lambda t: (t, 0)),
                   pl.BlockSpec((_N1, _CA), lambda t: (0, 0))),
        scratch_shapes=[pltpu.VMEM((_CA, _CA), jnp.float32)],
        compiler_params=pltpu.CompilerParams(
            dimension_semantics=("arbitrary",),
            vmem_limit_bytes=_VMEM),
    )(x2, w1fp, g1c, bt1c)

    y2, st2 = pl.pallas_call(
        _mid_kernel,
        out_shape=(jax.ShapeDtypeStruct((_N2, M), jnp.bfloat16),
                   jax.ShapeDtypeStruct((2 * _N2, 1), jnp.float32)),
        grid=(n_tiles,),
        in_specs=[
            pl.BlockSpec((_CA, TL), lambda t: (t // t_per_b, t % t_per_b)),
            pl.BlockSpec((_N1, _CA), lambda t: (0, 0)),
            pl.BlockSpec((_N2, _N1), lambda t: (0, 0)),
        ],
        out_specs=(pl.BlockSpec((_N2, TL), lambda t: (0, t)),
                   pl.BlockSpec((2 * _N2, 1), lambda t: (0, 0))),
        scratch_shapes=[pltpu.VMEM((_N2, 128), jnp.float32),
                        pltpu.VMEM((_N2, 128), jnp.float32)],
        compiler_params=pltpu.CompilerParams(
            dimension_semantics=("arbitrary",),
            vmem_limit_bytes=_VMEM),
    )(xpad, w1s, w2t)

    o2 = pl.pallas_call(
        functools.partial(_out_kernel, m_total=float(M), eps=_BN_EPS),
        out_shape=jax.ShapeDtypeStruct((B * _N3, L), jnp.float32),
        grid=(n_tiles,),
        in_specs=[
            pl.BlockSpec((_N2, TL), lambda t: (0, t)),
            pl.BlockSpec((2 * _N2, 1), lambda t: (0, 0)),
            pl.BlockSpec((2 * _N2 + _N3, 1), lambda t: (0, 0)),
            pl.BlockSpec((_N3, _N2), lambda t: (0, 0)),
        ],
        out_specs=pl.BlockSpec((_N3, TL),
                               lambda t: (t // t_per_b, t % t_per_b)),
        compiler_params=pltpu.CompilerParams(
            dimension_semantics=("arbitrary",),
            vmem_limit_bytes=_VMEM),
    )(y2, st2, pcol, w3t)

    return o2.reshape(B, _N3, L)


# speedup vs baseline: 1.4725x; 1.0052x over previous
import functools

import jax
import jax.numpy as jnp
from jax.experimental import pallas as pl
from jax.experimental.pallas import tpu as pltpu

_BN_EPS = 1e-5
_N1, _N2, _N3 = 1024, 512, 256
_CA = 40
_VMEM = 56 * 1024 * 1024


def _pre_kernel(x_ref, w1fp_ref, g1_ref, bt1_ref, xa_ref, w1s_ref, s_ref,
                *, m_total, eps):
    t = pl.program_id(0)
    n_t = pl.num_programs(0)
    ncols = x_ref.shape[1]
    nsub = x_ref.shape[0] // 32

    acc = None
    for i in range(nsub):
        xb = x_ref[32 * i:32 * (i + 1), :].astype(jnp.bfloat16)
        xa = jnp.concatenate(
            [xb, jnp.ones((1, ncols), jnp.bfloat16),
             jnp.zeros((7, ncols), jnp.bfloat16)], axis=0)
        xa_ref[_CA * i:_CA * (i + 1), :] = xa
        p = jax.lax.dot_general(
            xa, xa, (((1,), (1,)), ((), ())),
            preferred_element_type=jnp.float32)
        acc = p if acc is None else acc + p

    @pl.when(t == 0)
    def _():
        s_ref[...] = acc

    @pl.when(t > 0)
    def _():
        s_ref[...] += acc

    @pl.when(t == n_t - 1)
    def _():
        parts = s_ref[...]
        w1fp = w1fp_ref[...]
        inv_m = 1.0 / m_total
        msum = parts[32:33, :]
        es = jnp.sum(w1fp * (msum * inv_m), axis=1, keepdims=True)
        u = jax.lax.dot(w1fp, parts,
                        precision=jax.lax.Precision.HIGHEST,
                        preferred_element_type=jnp.float32)
        q = jnp.sum(u * w1fp, axis=1, keepdims=True)
        var = jnp.maximum(q * inv_m - es * es, 0.0)
        a1 = g1_ref[...] * jax.lax.rsqrt(var + eps)
        c1 = bt1_ref[...] - a1 * es
        lane = jax.lax.broadcasted_iota(jnp.int32, (_N1, _CA), 1)
        w1s_ref[...] = jnp.where(lane == 32, c1,
                                 w1fp * a1).astype(jnp.bfloat16)


def _mid_kernel(xa_ref, w1s_ref, w2_ref, y2_ref, st_ref, accs_ref, accq_ref):
    t = pl.program_id(0)
    n_t = pl.num_programs(0)

    y1 = jnp.dot(w1s_ref[...], xa_ref[...],
                 preferred_element_type=jnp.float32)
    h1 = jnp.maximum(y1, 0.0).astype(jnp.bfloat16)
    y2 = jnp.dot(w2_ref[...], h1, preferred_element_type=jnp.float32)
    y2_ref[...] = y2.astype(jnp.bfloat16)

    q2 = y2 * y2
    tl = y2.shape[1]
    ps = y2[:, 0:128]
    pq = q2[:, 0:128]
    for off in range(128, tl, 128):
        ps = ps + y2[:, off:off + 128]
        pq = pq + q2[:, off:off + 128]

    @pl.when(t == 0)
    def _():
        accs_ref[...] = ps
        accq_ref[...] = pq

    @pl.when(t > 0)
    def _():
        accs_ref[...] += ps
        accq_ref[...] += pq

    @pl.when(t == n_t - 1)
    def _():
        st_ref[0:_N2] = jnp.sum(accs_ref[...], axis=1, keepdims=True)
        st_ref[_N2:2 * _N2] = jnp.sum(accq_ref[...], axis=1, keepdims=True)


def _out_kernel(y2_ref, st_ref, p_ref, w3_ref, o_ref, *, m_total, eps):
    inv_m = 1.0 / m_total
    mean = st_ref[0:_N2] * inv_m
    var = jnp.maximum(st_ref[_N2:2 * _N2] * inv_m - mean * mean, 0.0)
    a2 = p_ref[0:_N2] * jax.lax.rsqrt(var + eps)
    c2 = p_ref[_N2:2 * _N2] - a2 * mean
    b3 = p_ref[2 * _N2:2 * _N2 + _N3]
    h2 = jnp.maximum(y2_ref[...].astype(jnp.float32) * a2 + c2,
                     0.0).astype(jnp.bfloat16)
    o_ref[...] = (jnp.dot(w3_ref[...], h2, preferred_element_type=jnp.float32)
                  + b3)


def kernel(x, w1, b1, w2, b2, w3, b3, g1, beta1, g2, beta2):
    B, Cin, L = x.shape
    M = B * L
    TL = 4096 if L % 4096 == 0 else (2048 if L % 2048 == 0 else L)
    n_tiles = M // TL
    t_per_b = L // TL

    x2 = x.reshape(B * Cin, L)

    w1fp = jnp.pad(w1.T, ((0, 0), (0, _CA - Cin)))
    w2t = w2.T.astype(jnp.bfloat16)
    w3t = w3.T.astype(jnp.bfloat16)
    g1c = g1.reshape(_N1, 1)
    bt1c = beta1.reshape(_N1, 1)
    pcol = jnp.concatenate(
        [g2, beta2, b3]).reshape(2 * _N2 + _N3, 1)

    bb = 8 if B % 8 == 0 else (4 if B % 4 == 0 else 1)
    xpad, w1s = pl.pallas_call(
        functools.partial(_pre_kernel, m_total=float(M), eps=_BN_EPS),
        out_shape=(jax.ShapeDtypeStruct((B * _CA, L), jnp.bfloat16),
                   jax.ShapeDtypeStruct((_N1, _CA), jnp.bfloat16)),
        grid=(B // bb,),
        in_specs=[
            pl.BlockSpec((bb * Cin, L), lambda t: (t, 0)),
            pl.BlockSpec((_N1, _CA), lambda t: (0, 0)),
            pl.BlockSpec((_N1, 1), lambda t: (0, 0)),
            pl.BlockSpec((_N1, 1), lambda t: (0, 0)),
        ],
        out_specs=(pl.BlockSpec((bb * _CA, L), lambda t: (t, 0)),
                   pl.BlockSpec((_N1, _CA), lambda t: (0, 0))),
        scratch_shapes=[pltpu.VMEM((_CA, _CA), jnp.float32)],
        compiler_params=pltpu.CompilerParams(
            dimension_semantics=("arbitrary",),
            vmem_limit_bytes=_VMEM),
    )(x2, w1fp, g1c, bt1c)

    y2, st2 = pl.pallas_call(
        _mid_kernel,
        out_shape=(jax.ShapeDtypeStruct((_N2, M), jnp.bfloat16),
                   jax.ShapeDtypeStruct((2 * _N2, 1), jnp.float32)),
        grid=(n_tiles,),
        in_specs=[
            pl.BlockSpec((_CA, TL), lambda t: (t // t_per_b, t % t_per_b)),
            pl.BlockSpec((_N1, _CA), lambda t: (0, 0)),
            pl.BlockSpec((_N2, _N1), lambda t: (0, 0)),
        ],
        out_specs=(pl.BlockSpec((_N2, TL), lambda t: (0, t)),
                   pl.BlockSpec((2 * _N2, 1), lambda t: (0, 0))),
        scratch_shapes=[pltpu.VMEM((_N2, 128), jnp.float32),
                        pltpu.VMEM((_N2, 128), jnp.float32)],
        compiler_params=pltpu.CompilerParams(
            dimension_semantics=("arbitrary",),
            vmem_limit_bytes=_VMEM),
    )(xpad, w1s, w2t)

    o2 = pl.pallas_call(
        functools.partial(_out_kernel, m_total=float(M), eps=_BN_EPS),
        out_shape=jax.ShapeDtypeStruct((B * _N3, L), jnp.float32),
        grid=(n_tiles,),
        in_specs=[
            pl.BlockSpec((_N2, TL), lambda t: (0, t)),
            pl.BlockSpec((2 * _N2, 1), lambda t: (0, 0)),
            pl.BlockSpec((2 * _N2 + _N3, 1), lambda t: (0, 0)),
            pl.BlockSpec((_N3, _N2), lambda t: (0, 0)),
        ],
        out_specs=pl.BlockSpec((_N3, TL),
                               lambda t: (t // t_per_b, t % t_per_b)),
        compiler_params=pltpu.CompilerParams(
            dimension_semantics=("arbitrary",),
            vmem_limit_bytes=_VMEM),
    )(y2, st2, pcol, w3t)

    return o2.reshape(B, _N3, L)
